# Initial kernel scaffold; baseline (speedup 1.0000x reference)
import jax
import jax.numpy as jnp
from jax.experimental import pallas as pl


def kernel(position_ids, table):
    raise NotImplementedError("write your pallas kernel here")


# speedup vs baseline: 1.9693x; 1.9693x over previous
import functools

import jax
import jax.numpy as jnp
from jax import lax
from jax.experimental import pallas as pl
from jax.experimental.pallas import tpu as pltpu
from jax.experimental.pallas import tpu_sc as plsc

D = 192
NC = 2
NS = 16
NW = NC * NS
CHUNK = 128
NBUF = 2


@functools.partial(jax.jit, static_argnames=("batch", "seq"))
def _lookup(ids_flat, table, *, batch, seq):
    B = batch * seq
    per_w = B // NW
    G = per_w // CHUNK

    ids3 = ids_flat.reshape(NW, G, CHUNK)

    mesh = plsc.VectorSubcoreMesh(core_axis_name="c", subcore_axis_name="s")

    @functools.partial(
        pl.kernel,
        out_type=jax.ShapeDtypeStruct((B, D), jnp.float32),
        mesh=mesh,
        compiler_params=pltpu.CompilerParams(use_tc_tiling_on_sc=False),
        scratch_types=[
            pltpu.VMEM((G, CHUNK), jnp.int32),
            pltpu.VMEM((CHUNK, D), jnp.float32),
            pltpu.VMEM((CHUNK, D), jnp.float32),
            pltpu.SemaphoreType.DMA,
            pltpu.SemaphoreType.DMA,
            pltpu.SemaphoreType.DMA,
            pltpu.SemaphoreType.DMA,
        ],
    )
    def k(ids_hbm, table_hbm, out_hbm, idx_v, row0, row1, g0, g1, w0, w1):
        wid = lax.axis_index("s") * NC + lax.axis_index("c")
        base = wid * per_w
        pltpu.sync_copy(ids_hbm.at[wid], idx_v)
        rows = (row0, row1)
        gsems = (g0, g1)
        wsems = (w0, w1)

        @pl.loop(0, G, step=NBUF)
        def _(gg):
            gathers = [
                pltpu.async_copy(
                    table_hbm.at[idx_v.at[gg + j]], rows[j], gsems[j]
                )
                for j in range(NBUF)
            ]
            writes = []
            for j in range(NBUF):
                gathers[j].wait()
                writes.append(
                    pltpu.async_copy(
                        rows[j],
                        out_hbm.at[pl.ds(base + (gg + j) * CHUNK, CHUNK)],
                        wsems[j],
                    )
                )
            for wcp in writes:
                wcp.wait()

    return k(ids3, table)


def kernel(position_ids, table):
    batch, seq = position_ids.shape
    ids_flat = position_ids.reshape(-1).astype(jnp.int32)
    out = _lookup(ids_flat, table, batch=batch, seq=seq)
    return out.reshape(batch, seq, D)


# speedup vs baseline: 2.0082x; 1.0198x over previous
import functools

import jax
import jax.numpy as jnp
from jax import lax
from jax.experimental import pallas as pl
from jax.experimental.pallas import tpu as pltpu
from jax.experimental.pallas import tpu_sc as plsc

V = 9217
D = 192
NC = 2
NS = 16
NW = NC * NS
CHUNK = 128
NBUF = 4
ROWS_PER_SUB = V // NS


@functools.partial(jax.jit, static_argnames=("batch", "seq"))
def _lookup(ids_flat, table, *, batch, seq):
    B = batch * seq
    per_w = B // NW
    G = per_w // CHUNK

    ids3 = ids_flat.reshape(NW, G, CHUNK)

    mesh = plsc.VectorSubcoreMesh(core_axis_name="c", subcore_axis_name="s")

    @functools.partial(
        pl.kernel,
        out_type=jax.ShapeDtypeStruct((B, D), jnp.float32),
        mesh=mesh,
        compiler_params=pltpu.CompilerParams(use_tc_tiling_on_sc=False),
        scratch_types=[
            pltpu.VMEM((G, CHUNK), jnp.int32),
            [pltpu.VMEM((CHUNK, D), jnp.float32) for _ in range(NBUF)],
            [pltpu.SemaphoreType.DMA for _ in range(NBUF)],
            [pltpu.SemaphoreType.DMA for _ in range(NBUF)],
        ],
    )
    def k(ids_hbm, table_hbm, out_hbm, idx_v, rows, gsems, wsems):
        sid = lax.axis_index("s")
        wid = sid * NC + lax.axis_index("c")
        base = wid * per_w
        shared_tab = table_hbm

        pltpu.sync_copy(ids_hbm.at[wid], idx_v)

        for j in range(2):
            pltpu.async_copy(shared_tab.at[idx_v.at[j]], rows[j], gsems[j])

        @pl.loop(0, G, step=NBUF)
        def _(g0):
            for j in range(NBUF):
                g = g0 + j
                b = j
                pltpu.make_async_copy(
                    shared_tab.at[idx_v.at[g]], rows[b], gsems[b]
                ).wait()
                pltpu.async_copy(
                    rows[b],
                    out_hbm.at[pl.ds(base + g * CHUNK, CHUNK)],
                    wsems[b],
                )
                b2 = (j + 2) % NBUF

                @pl.when(g >= 2)
                def _():
                    pltpu.make_async_copy(
                        rows[b2],
                        out_hbm.at[pl.ds(base + (g - 2) * CHUNK, CHUNK)],
                        wsems[b2],
                    ).wait()

                @pl.when(g + 2 < G)
                def _():
                    pltpu.async_copy(
                        shared_tab.at[idx_v.at[g + 2]], rows[b2], gsems[b2]
                    )

        for g in (G - 2, G - 1):
            b = g % NBUF
            pltpu.make_async_copy(
                rows[b],
                out_hbm.at[pl.ds(base + g * CHUNK, CHUNK)],
                wsems[b],
            ).wait()

    return k(ids3, table)


def kernel(position_ids, table):
    batch, seq = position_ids.shape
    ids_flat = position_ids.reshape(-1).astype(jnp.int32)
    out = _lookup(ids_flat, table, batch=batch, seq=seq)
    return out.reshape(batch, seq, D)


# speedup vs baseline: 2.1119x; 1.0516x over previous
import functools

import jax
import jax.numpy as jnp
from jax import lax
from jax.experimental import pallas as pl
from jax.experimental.pallas import tpu as pltpu
from jax.experimental.pallas import tpu_sc as plsc

V = 9217
D = 192
NC = 2
NS = 16
NW = NC * NS
CHUNK = 32
NBUF = 2
ROWS_PER_SUB = V // NS


@functools.partial(jax.jit, static_argnames=("batch", "seq"))
def _lookup(ids_flat, table, *, batch, seq):
    B = batch * seq
    per_w = B // NW
    G = per_w // CHUNK

    ids3 = ids_flat.reshape(NW, G, CHUNK)

    mesh = plsc.VectorSubcoreMesh(core_axis_name="c", subcore_axis_name="s")

    @functools.partial(
        pl.kernel,
        out_type=jax.ShapeDtypeStruct((B, D), jnp.float32),
        mesh=mesh,
        compiler_params=pltpu.CompilerParams(use_tc_tiling_on_sc=False),
        scratch_types=[
            pltpu.VMEM_SHARED((V, D), jnp.float32),
            pltpu.VMEM((G, CHUNK), jnp.int32),
            [pltpu.VMEM((CHUNK, D), jnp.float32) for _ in range(NBUF)],
            [pltpu.SemaphoreType.DMA for _ in range(NBUF)],
            [pltpu.SemaphoreType.DMA for _ in range(NBUF)],
        ],
    )
    def k(ids_hbm, table_hbm, out_hbm, shared_tab, idx_v, rows, gsems, wsems):
        sid = lax.axis_index("s")
        wid = sid * NC + lax.axis_index("c")
        base = wid * per_w

        pltpu.sync_copy(
            table_hbm.at[pl.ds(sid * ROWS_PER_SUB, ROWS_PER_SUB)],
            shared_tab.at[pl.ds(sid * ROWS_PER_SUB, ROWS_PER_SUB)],
        )

        @pl.when(sid == 0)
        def _():
            pltpu.sync_copy(
                table_hbm.at[pl.ds(NS * ROWS_PER_SUB, V - NS * ROWS_PER_SUB)],
                shared_tab.at[pl.ds(NS * ROWS_PER_SUB, V - NS * ROWS_PER_SUB)],
            )

        pltpu.sync_copy(ids_hbm.at[wid], idx_v)
        plsc.subcore_barrier()

        pltpu.async_copy(shared_tab.at[idx_v.at[0]], rows[0], gsems[0])

        @pl.loop(0, G, step=NBUF)
        def _(g0):
            for j in range(NBUF):
                g = g0 + j
                b = j
                pltpu.make_async_copy(
                    shared_tab.at[idx_v.at[g]], rows[b], gsems[b]
                ).wait()
                pltpu.async_copy(
                    rows[b],
                    out_hbm.at[pl.ds(base + g * CHUNK, CHUNK)],
                    wsems[b],
                )
                b2 = (j + 1) % NBUF

                @pl.when(g >= 1)
                def _():
                    pltpu.make_async_copy(
                        rows[b2],
                        out_hbm.at[pl.ds(base + (g - 1) * CHUNK, CHUNK)],
                        wsems[b2],
                    ).wait()

                @pl.when(g + 1 < G)
                def _():
                    pltpu.async_copy(
                        shared_tab.at[idx_v.at[g + 1]], rows[b2], gsems[b2]
                    )

        b = (G - 1) % NBUF
        pltpu.make_async_copy(
            rows[b],
            out_hbm.at[pl.ds(base + (G - 1) * CHUNK, CHUNK)],
            wsems[b],
        ).wait()

    return k(ids3, table)


def kernel(position_ids, table):
    batch, seq = position_ids.shape
    ids_flat = position_ids.reshape(-1).astype(jnp.int32)
    out = _lookup(ids_flat, table, batch=batch, seq=seq)
    return out.reshape(batch, seq, D)


# speedup vs baseline: 3.1317x; 1.4828x over previous
import functools

import jax
import jax.numpy as jnp
from jax import lax
from jax.experimental import pallas as pl
from jax.experimental.pallas import tpu as pltpu
from jax.experimental.pallas import tpu_sc as plsc

V = 9217
D = 192
DA = 128
DB = D - DA
NC = 2
NS = 16
NW = NC * NS
CHUNK = 128
NBUF = 2


@functools.partial(jax.jit, static_argnames=("batch", "seq"))
def _lookup(ids_flat, table, *, batch, seq):
    B = batch * seq
    per_w = B // NW
    G = per_w // CHUNK

    ids3 = ids_flat.reshape(NW, G, CHUNK)
    table_a = table[:, :DA]
    table_b = jnp.pad(table[:, DA:], ((0, 0), (0, DA - DB)))

    mesh = plsc.VectorSubcoreMesh(core_axis_name="c", subcore_axis_name="s")

    @functools.partial(
        pl.kernel,
        out_type=jax.ShapeDtypeStruct((B, D), jnp.float32),
        mesh=mesh,
        scratch_types=[
            pltpu.VMEM((G, CHUNK), jnp.int32),
            [pltpu.VMEM((CHUNK, DA), jnp.float32) for _ in range(NBUF)],
            [pltpu.VMEM((CHUNK, DA), jnp.float32) for _ in range(NBUF)],
            [pltpu.VMEM((CHUNK, DB), jnp.float32) for _ in range(NBUF)],
            [pltpu.SemaphoreType.DMA for _ in range(NBUF)],
            [pltpu.SemaphoreType.DMA for _ in range(NBUF)],
            [pltpu.SemaphoreType.DMA for _ in range(NBUF)],
            [pltpu.SemaphoreType.DMA for _ in range(NBUF)],
        ],
    )
    def k(ids_hbm, ta_hbm, tb_hbm, out_hbm, idx_v, bufa, bufb, bufb64,
          gsa, gsb, wsa, wsb):
        wid = lax.axis_index("s") * NC + lax.axis_index("c")
        base = wid * per_w

        pltpu.sync_copy(ids_hbm.at[wid], idx_v)

        def start_gather(g, b):
            pltpu.async_copy(ta_hbm.at[idx_v.at[g]], bufa[b], gsa[b])
            pltpu.async_copy(tb_hbm.at[idx_v.at[g]], bufb[b], gsb[b])

        def wait_gather(g, b):
            pltpu.make_async_copy(ta_hbm.at[idx_v.at[g]], bufa[b], gsa[b]).wait()
            pltpu.make_async_copy(tb_hbm.at[idx_v.at[g]], bufb[b], gsb[b]).wait()

        def compact_b(b):
            @pl.loop(0, CHUNK)
            def _(r):
                for c in range(DB // 16):
                    bufb64[b][r, pl.ds(c * 16, 16)] = (
                        bufb[b][r, pl.ds(c * 16, 16)]
                    )

        def start_write(g, b):
            r0 = base + g * CHUNK
            pltpu.async_copy(
                bufa[b], out_hbm.at[pl.ds(r0, CHUNK), pl.ds(0, DA)], wsa[b]
            )
            pltpu.async_copy(
                bufb64[b],
                out_hbm.at[pl.ds(r0, CHUNK), pl.ds(DA, DB)],
                wsb[b],
            )

        def wait_write(g, b):
            r0 = base + g * CHUNK
            pltpu.make_async_copy(
                bufa[b], out_hbm.at[pl.ds(r0, CHUNK), pl.ds(0, DA)], wsa[b]
            ).wait()
            pltpu.make_async_copy(
                bufb64[b],
                out_hbm.at[pl.ds(r0, CHUNK), pl.ds(DA, DB)],
                wsb[b],
            ).wait()

        start_gather(0, 0)

        @pl.loop(0, G, step=NBUF)
        def _(g0):
            for j in range(NBUF):
                g = g0 + j
                b = j
                wait_gather(g, b)
                compact_b(b)
                start_write(g, b)
                b2 = (j + 1) % NBUF

                @pl.when(g >= 1)
                def _():
                    wait_write(g - 1, b2)

                @pl.when(g + 1 < G)
                def _():
                    start_gather(g + 1, b2)

        wait_write(G - 1, (G - 1) % NBUF)

    return k(ids3, table_a, table_b)


def kernel(position_ids, table):
    batch, seq = position_ids.shape
    ids_flat = position_ids.reshape(-1).astype(jnp.int32)
    out = _lookup(ids_flat, table, batch=batch, seq=seq)
    return out.reshape(batch, seq, D)
